# baseline (speedup 1.0000x reference)
import functools

import jax
import jax.numpy as jnp
from jax import lax
from jax.experimental import pallas as pl
from jax.experimental.pallas import tpu as pltpu
from jax.experimental.pallas import tpu_sc as plsc

D = 64
BB = 128
BLOCKS_PER_W = 4
HIST = 50
UNITS = BLOCKS_PER_W * HIST


def _body(num_cores, table_hbm, xt_hbm, out_hbm, xblk_v,
          rows0_v, rows1_v, tile0_v, tile1_v, sem_g0, sem_g1):
    sems_g = (sem_g0, sem_g1)
    rows = (rows0_v, rows1_v)
    tiles = (tile0_v, tile1_v)
    wid = lax.axis_index("s") * num_cores + lax.axis_index("c")

    lane = lax.iota(jnp.int32, 16)

    def idx_ref(u):
        l = u % HIST
        k = u // HIST
        return xblk_v.at[l, pl.ds(k * BB, BB)]

    zeros16 = jnp.zeros((16,), jnp.int32)
    dh16 = [(16 * s + lane) // 8 for s in range(4)]
    dl16 = [(16 * s + lane) % 8 for s in range(4)]

    def transpose_unit(par):
        r = rows[par]
        t = tiles[par]

        @plsc.parallel_loop(0, BB, 1, unroll=8)
        def _(bl):
            blv = zeros16 + bl
            for s in range(4):
                vec = r[bl, pl.ds(16 * s, 16)]
                plsc.store_scatter(t, [dh16[s], dl16[s], blv], vec)

    def issue_gather(par, u):
        pltpu.async_copy(table_hbm.at[idx_ref(u)], rows[par], sems_g[par])

    def wait_gather(par):
        pltpu.make_async_copy(
            table_hbm.at[xblk_v.at[0, pl.ds(0, BB)]],
            rows[par], sems_g[par]).wait()

    pltpu.sync_copy(
        xt_hbm.at[:, pl.ds(wid * (BLOCKS_PER_W * BB), BLOCKS_PER_W * BB)],
        xblk_v)
    issue_gather(0, 0)

    def pairbody(i, carry):
        for par in (0, 1):
            u = 2 * i + par
            wait_gather(par)

            @pl.when(u < UNITS - 1)
            def _():
                issue_gather(1 - par, u + 1)

            transpose_unit(par)
            l = u % HIST
            bh = wid * BLOCKS_PER_W + u // HIST
            pltpu.sync_copy(tiles[par].at[:, :, pl.ds(0, BB)],
                            out_hbm.at[l, :, bh])
        return carry

    lax.fori_loop(0, UNITS // 2, pairbody, 0)


def kernel(x, embed_table):
    B, H = x.shape
    info = plsc.get_sparse_core_info()
    nw = info.num_cores * info.num_subcores
    assert nw * BLOCKS_PER_W * BB == B and H == HIST

    xt = x.T.astype(jnp.int32)
    tab_pad = jnp.pad(embed_table, ((0, 0), (0, D)))
    mesh = plsc.VectorSubcoreMesh(core_axis_name="c", subcore_axis_name="s")

    gather = functools.partial(
        pl.kernel,
        mesh=mesh,
        out_type=jax.ShapeDtypeStruct((HIST, 8, B // BB, 8, BB), jnp.float32),
        scratch_types=[
            pltpu.VMEM((HIST, BLOCKS_PER_W * BB), jnp.int32),
            pltpu.VMEM((BB, 2 * D), jnp.float32),
            pltpu.VMEM((BB, 2 * D), jnp.float32),
            pltpu.VMEM((8, 8, BB + 1), jnp.float32),
            pltpu.VMEM((8, 8, BB + 1), jnp.float32),
        ] + [pltpu.SemaphoreType.DMA] * 2,
        compiler_params=pltpu.CompilerParams(
            use_tc_tiling_on_sc=False, needs_layout_passes=False),
    )(functools.partial(_body, info.num_cores))

    out5 = gather(tab_pad, xt)
    return out5.transpose(2, 4, 0, 1, 3).reshape(B, H, D)


# speedup vs baseline: 1.0215x; 1.0215x over previous
import functools

import jax
import jax.numpy as jnp
from jax import lax
from jax.experimental import pallas as pl
from jax.experimental.pallas import tpu as pltpu
from jax.experimental.pallas import tpu_sc as plsc

D = 64
BB = 128
BLOCKS_PER_W = 4
HIST = 50
UNITS = BLOCKS_PER_W * HIST


def _body(num_cores, table_hbm, xt_hbm, out_hbm, xblk_v,
          rows0_v, rows1_v, tile0_v, tile1_v, sem_g0, sem_g1):
    sems_g = (sem_g0, sem_g1)
    rows = (rows0_v, rows1_v)
    tiles = (tile0_v, tile1_v)
    wid = lax.axis_index("s") * num_cores + lax.axis_index("c")

    lane = lax.iota(jnp.int32, 16)

    def idx_ref(u):
        l = u % HIST
        k = u // HIST
        return xblk_v.at[l, pl.ds(k * BB, BB)]

    zeros16 = jnp.zeros((16,), jnp.int32)
    dh16 = [(16 * s + lane) // 8 for s in range(4)]
    dl16 = [(16 * s + lane) % 8 for s in range(4)]

    def transpose_unit(par):
        r = rows[par]
        t = tiles[par]

        @plsc.parallel_loop(0, BB, 1, unroll=8)
        def _(bl):
            blv = zeros16 + bl
            for s in range(4):
                vec = r[bl, pl.ds(16 * s, 16)]
                plsc.store_scatter(t, [dh16[s], dl16[s], blv], vec)

    def issue_gather(par, u):
        pltpu.async_copy(table_hbm.at[idx_ref(u)], rows[par], sems_g[par])

    def wait_gather(par):
        pltpu.make_async_copy(
            table_hbm.at[xblk_v.at[0, pl.ds(0, BB)]],
            rows[par], sems_g[par]).wait()

    pltpu.sync_copy(
        xt_hbm.at[:, pl.ds(wid * (BLOCKS_PER_W * BB), BLOCKS_PER_W * BB)],
        xblk_v)
    issue_gather(0, 0)

    def pairbody(i, carry):
        for par in (0, 1):
            u = 2 * i + par
            wait_gather(par)

            @pl.when(u < UNITS - 1)
            def _():
                issue_gather(1 - par, u + 1)

            transpose_unit(par)
            l = u % HIST
            bh = wid * BLOCKS_PER_W + u // HIST
            pltpu.sync_copy(tiles[par].at[:, :, pl.ds(0, BB)],
                            out_hbm.at[l, :, bh])
        return carry

    lax.fori_loop(0, UNITS // 2, pairbody, 0)


def kernel(x, embed_table):
    B, H = x.shape
    info = plsc.get_sparse_core_info()
    nw = info.num_cores * info.num_subcores
    assert nw * BLOCKS_PER_W * BB == B and H == HIST

    xt = x.T.astype(jnp.int32)
    mesh = plsc.VectorSubcoreMesh(core_axis_name="c", subcore_axis_name="s")

    gather = functools.partial(
        pl.kernel,
        mesh=mesh,
        out_type=jax.ShapeDtypeStruct((HIST, 8, B // BB, 8, BB), jnp.float32),
        scratch_types=[
            pltpu.VMEM((HIST, BLOCKS_PER_W * BB), jnp.int32),
            pltpu.VMEM((BB, D), jnp.float32),
            pltpu.VMEM((BB, D), jnp.float32),
            pltpu.VMEM((8, 8, BB + 1), jnp.float32),
            pltpu.VMEM((8, 8, BB + 1), jnp.float32),
        ] + [pltpu.SemaphoreType.DMA] * 2,
        compiler_params=pltpu.CompilerParams(
            use_tc_tiling_on_sc=False, needs_layout_passes=False),
    )(functools.partial(_body, info.num_cores))

    out5 = gather(embed_table, xt)
    return out5.transpose(2, 4, 0, 1, 3).reshape(B, H, D)
